# baseline (speedup 1.0000x reference)
import functools

import jax
import jax.numpy as jnp
from jax import lax
from jax.experimental import pallas as pl
from jax.experimental.pallas import tpu as pltpu

N = 10000
E = 320000
EMB = 128
NP = 10240
BE = 2560
EP2 = 643072


def _tables_body(xp, hl, nt, whi, wlo, bias, out):
    xb = xp[...]
    m = jnp.max(jnp.abs(xb), axis=1, keepdims=True) == 0.0
    xe = jnp.where(m, nt[...][0][None, :], xb)
    acc = jnp.dot(xe, whi[0], preferred_element_type=jnp.float32)
    acc += jnp.dot(hl[...], wlo[0], preferred_element_type=jnp.float32)
    out[...] = (acc + bias[...][0])[None]


def _edge_body(qb, ab, gab, gbb, wq, wa, w2, b2, out):
    z = jnp.dot(qb[...], wq[...], preferred_element_type=jnp.float32)
    z += jnp.dot(ab[...], wa[...], preferred_element_type=jnp.float32)
    z += gab[...] + gbb[...]
    h = jnp.maximum(z, 0.0)
    o = jnp.dot(h, w2[...], preferred_element_type=jnp.float32)
    out[...] = o + b2[...][0][None, :]


def _build_tables(xp, hl, non_text_emb, whi, wlo, bias):
    RB = 1280
    return pl.pallas_call(
        _tables_body,
        grid=(2, NP // RB),
        in_specs=[
            pl.BlockSpec((RB, EMB), lambda h, i: (i, 0)),
            pl.BlockSpec((RB, 16), lambda h, i: (i, 0)),
            pl.BlockSpec((1, EMB), lambda h, i: (0, 0)),
            pl.BlockSpec((1, EMB, EMB), lambda h, i: (h, 0, 0)),
            pl.BlockSpec((1, 16, EMB), lambda h, i: (h, 0, 0)),
            pl.BlockSpec((1, 1, EMB), lambda h, i: (h, 0, 0)),
        ],
        out_specs=pl.BlockSpec((1, RB, EMB), lambda h, i: (h, i, 0)),
        out_shape=jax.ShapeDtypeStruct((2, NP, EMB), jnp.float32),
    )(xp, hl, non_text_emb, whi, wlo, bias)


def _edge_mlp(q_emb, edge_attr, g, wq, wa, w2, b2):
    nblk = E // BE
    return pl.pallas_call(
        _edge_body,
        grid=(nblk,),
        in_specs=[
            pl.BlockSpec((BE, EMB), lambda i: (i, 0)),
            pl.BlockSpec((BE, EMB), lambda i: (i, 0)),
            pl.BlockSpec((BE, EMB), lambda i: (i, 0)),
            pl.BlockSpec((BE, EMB), lambda i: (E // BE + i, 0)),
            pl.BlockSpec((EMB, EMB), lambda i: (0, 0)),
            pl.BlockSpec((EMB, EMB), lambda i: (0, 0)),
            pl.BlockSpec((EMB, 1), lambda i: (0, 0)),
            pl.BlockSpec((1, 1), lambda i: (0, 0)),
        ],
        out_specs=pl.BlockSpec((BE, 1), lambda i: (i, 0)),
        out_shape=jax.ShapeDtypeStruct((E, 1), jnp.float32),
    )(q_emb, edge_attr, g, g, wq, wa, w2, b2)


def _dde(topic, src, dst):
    def conv(h, s, d):
        msg = jnp.take(h, s, axis=0)
        agg = jax.ops.segment_sum(msg, d, num_segments=N)
        deg = jax.ops.segment_sum(jnp.ones((E, 1), jnp.float32), d, num_segments=N)
        return agg / jnp.maximum(deg, 1.0)

    h1 = conv(topic, src, dst)
    h2 = conv(h1, src, dst)
    r1 = conv(topic, dst, src)
    r2 = conv(r1, dst, src)
    return jnp.concatenate([h1, h2, r1, r2], axis=1)


def kernel(x, edge_index, edge_attr, topic_signal, q_emb, non_text_emb, W1, b1, W2, b2):
    src = edge_index[0]
    dst = edge_index[1]

    dde = _dde(topic_signal, src, dst)
    hl = jnp.concatenate([topic_signal, dde], axis=1)
    hl = jnp.pad(hl, ((0, NP - N), (0, 6)))
    xp = jnp.pad(x, ((0, NP - N), (0, 0)))

    whi = jnp.stack([W1[128:256], W1[394:522]])
    wlo = jnp.stack([jnp.pad(W1[256:266], ((0, 6), (0, 0))),
                     jnp.pad(W1[522:532], ((0, 6), (0, 0)))])
    bias = jnp.stack([b1, jnp.zeros_like(b1)])[:, None, :]

    tables = _build_tables(xp, hl, non_text_emb, whi, wlo, bias)
    t2 = tables.reshape(2 * NP, EMB)

    idx = jnp.concatenate([src, dst + NP])
    idx = jnp.pad(idx, (0, EP2 - 2 * E))
    g = jnp.take(t2, idx, axis=0)

    out = _edge_mlp(q_emb, edge_attr, g, W1[0:128], W1[266:394], W2, b2.reshape(1, 1))
    return out.reshape(E)


# speedup vs baseline: 1.2595x; 1.2595x over previous
import functools

import jax
import jax.numpy as jnp
from jax import lax
from jax.experimental import pallas as pl
from jax.experimental.pallas import tpu as pltpu
from jax.experimental.pallas import tpu_sc as plsc

N = 10000
E = 320000
EMB = 128
NP = 10240
BE = 2560
NSC, NSUB = 2, 16
NW = NSC * NSUB
CH = 128
NCH = 158
BPW = CH * NCH
EP2 = NW * BPW


def _tables_body(xp, hl, nt, whi, wlo, bias, out):
    xb = xp[...]
    m = jnp.max(jnp.abs(xb), axis=1, keepdims=True) == 0.0
    xe = jnp.where(m, nt[...][0][None, :], xb)
    acc = jnp.dot(xe, whi[0], preferred_element_type=jnp.float32)
    acc += jnp.dot(hl[...], wlo[0], preferred_element_type=jnp.float32)
    out[...] = (acc + bias[...][0])[None]


def _edge_body(qb, ab, gab, gbb, wq, wa, w2, b2, out):
    z = jnp.dot(qb[...], wq[...], preferred_element_type=jnp.float32)
    z += jnp.dot(ab[...], wa[...], preferred_element_type=jnp.float32)
    z += gab[...] + gbb[...]
    h = jnp.maximum(z, 0.0)
    o = jnp.dot(h, w2[...], preferred_element_type=jnp.float32)
    out[...] = o + b2[...][0][None, :]


def _build_tables(xp, hl, non_text_emb, whi, wlo, bias):
    RB = 1280
    return pl.pallas_call(
        _tables_body,
        grid=(2, NP // RB),
        in_specs=[
            pl.BlockSpec((RB, EMB), lambda h, i: (i, 0)),
            pl.BlockSpec((RB, 16), lambda h, i: (i, 0)),
            pl.BlockSpec((1, EMB), lambda h, i: (0, 0)),
            pl.BlockSpec((1, EMB, EMB), lambda h, i: (h, 0, 0)),
            pl.BlockSpec((1, 16, EMB), lambda h, i: (h, 0, 0)),
            pl.BlockSpec((1, 1, EMB), lambda h, i: (h, 0, 0)),
        ],
        out_specs=pl.BlockSpec((1, RB, EMB), lambda h, i: (h, i, 0)),
        out_shape=jax.ShapeDtypeStruct((2, NP, EMB), jnp.float32),
    )(xp, hl, non_text_emb, whi, wlo, bias)


def _edge_mlp(q_emb, edge_attr, g, wq, wa, w2, b2):
    nblk = E // BE
    return pl.pallas_call(
        _edge_body,
        grid=(nblk,),
        in_specs=[
            pl.BlockSpec((BE, EMB), lambda i: (i, 0)),
            pl.BlockSpec((BE, EMB), lambda i: (i, 0)),
            pl.BlockSpec((BE, EMB), lambda i: (i, 0)),
            pl.BlockSpec((BE, EMB), lambda i: (E // BE + i, 0)),
            pl.BlockSpec((EMB, EMB), lambda i: (0, 0)),
            pl.BlockSpec((EMB, EMB), lambda i: (0, 0)),
            pl.BlockSpec((EMB, 1), lambda i: (0, 0)),
            pl.BlockSpec((1, 1), lambda i: (0, 0)),
        ],
        out_specs=pl.BlockSpec((BE, 1), lambda i: (i, 0)),
        out_shape=jax.ShapeDtypeStruct((E, 1), jnp.float32),
    )(q_emb, edge_attr, g, g, wq, wa, w2, b2)


_SC_MESH = plsc.VectorSubcoreMesh(
    core_axis_name="c", subcore_axis_name="s", num_cores=NSC, num_subcores=NSUB)


@functools.partial(
    pl.kernel,
    out_type=jax.ShapeDtypeStruct((EP2, EMB), jnp.float32),
    mesh=_SC_MESH,
    scratch_types=[
        pltpu.VMEM((BPW,), jnp.int32),
        pltpu.VMEM((CH, EMB), jnp.float32),
        pltpu.VMEM((CH, EMB), jnp.float32),
        pltpu.SemaphoreType.DMA,
        pltpu.SemaphoreType.DMA,
    ],
)
def _sc_gather(t_hbm, idx_hbm, out_hbm, idx_v, buf0, buf1, sem0, sem1):
    wid = lax.axis_index("s") * NSC + lax.axis_index("c")
    base = wid * BPW
    pltpu.sync_copy(idx_hbm.at[pl.ds(base, BPW)], idx_v)
    pltpu.async_copy(t_hbm.at[idx_v.at[pl.ds(0, CH)]], buf0, sem0)

    def pair(k):
        j0 = 2 * k
        pltpu.async_copy(t_hbm.at[idx_v.at[pl.ds((j0 + 1) * CH, CH)]], buf1, sem1)
        pltpu.make_async_copy(t_hbm.at[pl.ds(0, CH)], buf0, sem0).wait()
        pltpu.sync_copy(buf0, out_hbm.at[pl.ds(base + j0 * CH, CH)])

        @pl.when(j0 + 2 < NCH)
        def _():
            pltpu.async_copy(t_hbm.at[idx_v.at[pl.ds((j0 + 2) * CH, CH)]], buf0, sem0)

        pltpu.make_async_copy(t_hbm.at[pl.ds(0, CH)], buf1, sem1).wait()
        pltpu.sync_copy(buf1, out_hbm.at[pl.ds(base + (j0 + 1) * CH, CH)])

    pl.loop(0, NCH // 2)(pair)


def _dde(topic, src, dst):
    def conv(h, s, d):
        msg = jnp.take(h, s, axis=0)
        agg = jax.ops.segment_sum(msg, d, num_segments=N)
        deg = jax.ops.segment_sum(jnp.ones((E, 1), jnp.float32), d, num_segments=N)
        return agg / jnp.maximum(deg, 1.0)

    h1 = conv(topic, src, dst)
    h2 = conv(h1, src, dst)
    r1 = conv(topic, dst, src)
    r2 = conv(r1, dst, src)
    return jnp.concatenate([h1, h2, r1, r2], axis=1)


def kernel(x, edge_index, edge_attr, topic_signal, q_emb, non_text_emb, W1, b1, W2, b2):
    src = edge_index[0]
    dst = edge_index[1]

    dde = _dde(topic_signal, src, dst)
    hl = jnp.concatenate([topic_signal, dde], axis=1)
    hl = jnp.pad(hl, ((0, NP - N), (0, 6)))
    xp = jnp.pad(x, ((0, NP - N), (0, 0)))

    whi = jnp.stack([W1[128:256], W1[394:522]])
    wlo = jnp.stack([jnp.pad(W1[256:266], ((0, 6), (0, 0))),
                     jnp.pad(W1[522:532], ((0, 6), (0, 0)))])
    bias = jnp.stack([b1, jnp.zeros_like(b1)])[:, None, :]

    tables = _build_tables(xp, hl, non_text_emb, whi, wlo, bias)
    t2 = tables.reshape(2 * NP, EMB)

    idx = jnp.concatenate([src, dst + NP])
    idx = jnp.pad(idx, (0, EP2 - 2 * E))
    g = _sc_gather(t2, idx)

    out = _edge_mlp(q_emb, edge_attr, g, W1[0:128], W1[266:394], W2, b2.reshape(1, 1))
    return out.reshape(E)


# speedup vs baseline: 8.9723x; 7.1235x over previous
import functools

import jax
import jax.numpy as jnp
from jax import lax
from jax.experimental import pallas as pl
from jax.experimental.pallas import tpu as pltpu
from jax.experimental.pallas import tpu_sc as plsc

N = 10000
E = 320000
EMB = 128
NP = 10240
BE = 2560
NSC, NSUB = 2, 16
NW = NSC * NSUB
CH = 128
NCH = 158
BPW = CH * NCH
EP2 = NW * BPW


def _tables_body(xp, hl, nt, whi, wlo, bias, out):
    xb = xp[...]
    m = jnp.max(jnp.abs(xb), axis=1, keepdims=True) == 0.0
    xe = jnp.where(m, nt[...][0][None, :], xb)
    acc = jnp.dot(xe, whi[0], preferred_element_type=jnp.float32)
    acc += jnp.dot(hl[...], wlo[0], preferred_element_type=jnp.float32)
    out[...] = (acc + bias[...][0])[None]


def _edge_body(qb, ab, gab, gbb, wq, wa, w2, b2, out):
    z = jnp.dot(qb[...], wq[...], preferred_element_type=jnp.float32)
    z += jnp.dot(ab[...], wa[...], preferred_element_type=jnp.float32)
    z += gab[...] + gbb[...]
    h = jnp.maximum(z, 0.0)
    o = jnp.dot(h, w2[...], preferred_element_type=jnp.float32)
    out[...] = o + b2[...][0][None, :]


def _build_tables(xp, hl, non_text_emb, whi, wlo, bias):
    RB = 1280
    return pl.pallas_call(
        _tables_body,
        grid=(2, NP // RB),
        in_specs=[
            pl.BlockSpec((RB, EMB), lambda h, i: (i, 0)),
            pl.BlockSpec((RB, 16), lambda h, i: (i, 0)),
            pl.BlockSpec((1, EMB), lambda h, i: (0, 0)),
            pl.BlockSpec((1, EMB, EMB), lambda h, i: (h, 0, 0)),
            pl.BlockSpec((1, 16, EMB), lambda h, i: (h, 0, 0)),
            pl.BlockSpec((1, 1, EMB), lambda h, i: (h, 0, 0)),
        ],
        out_specs=pl.BlockSpec((1, RB, EMB), lambda h, i: (h, i, 0)),
        out_shape=jax.ShapeDtypeStruct((2, NP, EMB), jnp.float32),
    )(xp, hl, non_text_emb, whi, wlo, bias)


def _edge_mlp(q_emb, edge_attr, g, wq, wa, w2, b2):
    nblk = E // BE
    return pl.pallas_call(
        _edge_body,
        grid=(nblk,),
        in_specs=[
            pl.BlockSpec((BE, EMB), lambda i: (i, 0)),
            pl.BlockSpec((BE, EMB), lambda i: (i, 0)),
            pl.BlockSpec((BE, EMB), lambda i: (i, 0)),
            pl.BlockSpec((BE, EMB), lambda i: (E // BE + i, 0)),
            pl.BlockSpec((EMB, EMB), lambda i: (0, 0)),
            pl.BlockSpec((EMB, EMB), lambda i: (0, 0)),
            pl.BlockSpec((EMB, 1), lambda i: (0, 0)),
            pl.BlockSpec((1, 1), lambda i: (0, 0)),
        ],
        out_specs=pl.BlockSpec((BE, 1), lambda i: (i, 0)),
        out_shape=jax.ShapeDtypeStruct((E, 1), jnp.float32),
    )(q_emb, edge_attr, g, g, wq, wa, w2, b2)


_SC_MESH = plsc.VectorSubcoreMesh(
    core_axis_name="c", subcore_axis_name="s", num_cores=NSC, num_subcores=NSUB)


@functools.partial(
    pl.kernel,
    out_type=jax.ShapeDtypeStruct((EP2, EMB), jnp.float32),
    mesh=_SC_MESH,
    scratch_types=[
        pltpu.VMEM((BPW,), jnp.int32),
        pltpu.VMEM((CH, EMB), jnp.float32),
        pltpu.VMEM((CH, EMB), jnp.float32),
        pltpu.SemaphoreType.DMA,
        pltpu.SemaphoreType.DMA,
    ],
)
def _sc_gather(t_hbm, idx_hbm, out_hbm, idx_v, buf0, buf1, sem0, sem1):
    wid = lax.axis_index("s") * NSC + lax.axis_index("c")
    base = wid * BPW
    pltpu.sync_copy(idx_hbm.at[pl.ds(base, BPW)], idx_v)
    pltpu.async_copy(t_hbm.at[idx_v.at[pl.ds(0, CH)]], buf0, sem0)

    def pair(k):
        j0 = 2 * k
        pltpu.async_copy(t_hbm.at[idx_v.at[pl.ds((j0 + 1) * CH, CH)]], buf1, sem1)
        pltpu.make_async_copy(t_hbm.at[pl.ds(0, CH)], buf0, sem0).wait()
        pltpu.sync_copy(buf0, out_hbm.at[pl.ds(base + j0 * CH, CH)])

        @pl.when(j0 + 2 < NCH)
        def _():
            pltpu.async_copy(t_hbm.at[idx_v.at[pl.ds((j0 + 2) * CH, CH)]], buf0, sem0)

        pltpu.make_async_copy(t_hbm.at[pl.ds(0, CH)], buf1, sem1).wait()
        pltpu.sync_copy(buf1, out_hbm.at[pl.ds(base + (j0 + 1) * CH, CH)])

    pl.loop(0, NCH // 2)(pair)


ECH = 128
ENCH = 157
ETW = ECH * ENCH
EPAD = NSUB * ETW
NSL = NP // NSUB


@functools.partial(
    pl.kernel,
    out_type=jax.ShapeDtypeStruct((8, NP), jnp.float32),
    mesh=_SC_MESH,
    compiler_params=pltpu.CompilerParams(needs_layout_passes=False),
    scratch_types=[
        pltpu.VMEM((ETW,), jnp.int32),
        pltpu.VMEM((ENCH, ECH), jnp.int32),
        pltpu.VMEM((ECH,), jnp.float32),
        pltpu.VMEM((ECH,), jnp.float32),
        pltpu.VMEM((ECH,), jnp.float32),
        pltpu.VMEM((NSL,), jnp.float32),
        pltpu.VMEM((NSL,), jnp.float32),
        pltpu.VMEM((NSL,), jnp.float32),
        pltpu.VMEM((NSL,), jnp.float32),
        pltpu.VMEM((NSL,), jnp.float32),
        pltpu.VMEM((NP,), jnp.float32),
        pltpu.VMEM((NP,), jnp.float32),
        pltpu.VMEM((NSL,), jnp.float32),
        pltpu.VMEM((NSL,), jnp.float32),
        pltpu.VMEM_SHARED((NP,), jnp.float32),
        pltpu.VMEM_SHARED((NP,), jnp.float32),
        pltpu.VMEM_SHARED((NP,), jnp.float32),
        pltpu.VMEM_SHARED((NP,), jnp.float32),
        pltpu.VMEM_SHARED((NP,), jnp.float32),
    ],
)
def _sc_dde(srcf, dstf, srcr, dstr, topic, out, sv, dv, m0v, m1v, ones_v,
            zsl, a0v, a1v, dgv, dmv, h0v, h1v, h0s, h1s,
            agg0_sh, agg1_sh, deg_sh, h0_sh, h1_sh):
    cid = lax.axis_index("c")
    sid = lax.axis_index("s")
    zf = jnp.zeros((16,), jnp.float32)
    onef = jnp.full((16,), 1.0, jnp.float32)

    @pl.when(cid == 0)
    def _():
        pltpu.sync_copy(srcf.at[pl.ds(sid * ETW, ETW)], sv)
        pltpu.sync_copy(dstr.at[sid], dv)

    @pl.when(cid == 1)
    def _():
        pltpu.sync_copy(dstf.at[pl.ds(sid * ETW, ETW)], sv)
        pltpu.sync_copy(srcr.at[sid], dv)

    pltpu.sync_copy(topic.at[0], h0v)
    pltpu.sync_copy(topic.at[1], h1v)

    for b in range(ECH // 16):
        ones_v[pl.ds(16 * b, 16)] = onef

    def zr(i):
        zsl[pl.ds(16 * i, 16)] = zf
    pl.loop(0, NSL // 16)(zr)
    nb = sid * NSL
    sl = pl.ds(nb, NSL)
    pltpu.sync_copy(zsl, agg0_sh.at[sl])
    pltpu.sync_copy(zsl, agg1_sh.at[sl])
    pltpu.sync_copy(zsl, deg_sh.at[sl])
    plsc.subcore_barrier()

    for r in range(2):
        def chunk(j):
            for b in range(ECH // 16):
                s16 = sv[pl.ds(j * ECH + 16 * b, 16)]
                m0v[pl.ds(16 * b, 16)] = plsc.load_gather(h0v, [s16])
                m1v[pl.ds(16 * b, 16)] = plsc.load_gather(h1v, [s16])
            pltpu.sync_copy(m0v, agg0_sh.at[dv.at[j]], add=True)
            pltpu.sync_copy(m1v, agg1_sh.at[dv.at[j]], add=True)
            if r == 0:
                pltpu.sync_copy(ones_v, deg_sh.at[dv.at[j]], add=True)
        pl.loop(0, ENCH)(chunk)
        plsc.subcore_barrier()

        pltpu.sync_copy(agg0_sh.at[sl], a0v)
        pltpu.sync_copy(agg1_sh.at[sl], a1v)
        pltpu.sync_copy(zsl, agg0_sh.at[sl])
        pltpu.sync_copy(zsl, agg1_sh.at[sl])
        if r == 0:
            pltpu.sync_copy(deg_sh.at[sl], dgv)

            def dmx(i):
                dmv[pl.ds(16 * i, 16)] = jnp.maximum(dgv[pl.ds(16 * i, 16)], 1.0)
            pl.loop(0, NSL // 16)(dmx)

        def red(i):
            s = pl.ds(16 * i, 16)
            h0s[s] = a0v[s] / dmv[s]
            h1s[s] = a1v[s] / dmv[s]
        pl.loop(0, NSL // 16)(red)
        pltpu.sync_copy(h0s, h0_sh.at[sl])
        pltpu.sync_copy(h1s, h1_sh.at[sl])
        row0 = cid * 4 + 2 * r
        pltpu.sync_copy(h0s, out.at[row0, sl])
        pltpu.sync_copy(h1s, out.at[row0 + 1, sl])
        plsc.subcore_barrier()
        pltpu.sync_copy(h0_sh, h0v)
        pltpu.sync_copy(h1_sh, h1v)


def _dde(topic, src, dst):
    pad = EPAD - E
    srcp = jnp.concatenate([src, jnp.full((pad,), NP - 1, jnp.int32)])
    dstp = jnp.concatenate([dst, jnp.full((pad,), NP - 1, jnp.int32)])
    topic_pl = jnp.pad(topic, ((0, NP - N), (0, 0))).T
    dde_pl = _sc_dde(srcp, dstp, srcp.reshape(NSUB, ENCH, ECH),
                     dstp.reshape(NSUB, ENCH, ECH), topic_pl)
    return dde_pl[:, :N].T


def kernel(x, edge_index, edge_attr, topic_signal, q_emb, non_text_emb, W1, b1, W2, b2):
    src = edge_index[0]
    dst = edge_index[1]

    dde = _dde(topic_signal, src, dst)
    hl = jnp.concatenate([topic_signal, dde], axis=1)
    hl = jnp.pad(hl, ((0, NP - N), (0, 6)))
    xp = jnp.pad(x, ((0, NP - N), (0, 0)))

    whi = jnp.stack([W1[128:256], W1[394:522]])
    wlo = jnp.stack([jnp.pad(W1[256:266], ((0, 6), (0, 0))),
                     jnp.pad(W1[522:532], ((0, 6), (0, 0)))])
    bias = jnp.stack([b1, jnp.zeros_like(b1)])[:, None, :]

    tables = _build_tables(xp, hl, non_text_emb, whi, wlo, bias)
    t2 = tables.reshape(2 * NP, EMB)

    idx = jnp.concatenate([src, dst + NP])
    idx = jnp.pad(idx, (0, EP2 - 2 * E))
    g = _sc_gather(t2, idx)

    out = _edge_mlp(q_emb, edge_attr, g, W1[0:128], W1[266:394], W2, b2.reshape(1, 1))
    return out.reshape(E)
